# Initial kernel scaffold; baseline (speedup 1.0000x reference)
import jax
import jax.numpy as jnp
from jax.experimental import pallas as pl


def kernel(x, r_ij, r_ik, r_jk, neighbors_j, triple_masks, W_in2f, W_fd, b_fd, W_ft, b_ft, W_out, b_out):
    raise NotImplementedError("write your pallas kernel here")


# speedup vs baseline: 3356.8300x; 3356.8300x over previous
import functools

import jax
import jax.numpy as jnp
from jax import lax
from jax.experimental import pallas as pl
from jax.experimental.pallas import tpu as pltpu
from jax.experimental.pallas import tpu_sc as plsc

N_B, N_A, N_NBH = 2, 10000, 16
N_IN, N_FILTERS, N_OUT = 128, 64, 128
N_ZETA = 3
_C1, _C2, _C3 = 1.0, 2.0 ** (-3.5), 2.0 ** (-7.0)

A_BLK_A = 1000
A_BLK_C = 400

SC_NC, SC_NS = 2, 16
E_TOTAL = N_B * N_A * N_NBH
E_PER_W = E_TOTAL // (SC_NC * SC_NS)
CHE = 400
STEPS = E_PER_W // CHE


def _a_body(x_ref, nbh_ref, w_ref, y_ref, gidx_ref):
    b = pl.program_id(0)
    y_ref[0] = jnp.dot(x_ref[0], w_ref[...], preferred_element_type=jnp.float32)
    gidx_ref[0] = nbh_ref[0] + b * N_A


def _stage_a(x, nbh, W_in2f):
    grid = (N_B, N_A // A_BLK_A)
    return pl.pallas_call(
        _a_body,
        grid=grid,
        in_specs=[
            pl.BlockSpec((1, A_BLK_A, N_IN), lambda b, i: (b, i, 0)),
            pl.BlockSpec((1, A_BLK_A, N_NBH), lambda b, i: (b, i, 0)),
            pl.BlockSpec((N_IN, 2 * N_FILTERS), lambda b, i: (0, 0)),
        ],
        out_specs=[
            pl.BlockSpec((1, A_BLK_A, 2 * N_FILTERS), lambda b, i: (b, i, 0)),
            pl.BlockSpec((1, A_BLK_A, N_NBH), lambda b, i: (b, i, 0)),
        ],
        out_shape=[
            jax.ShapeDtypeStruct((N_B, N_A, 2 * N_FILTERS), jnp.float32),
            jax.ShapeDtypeStruct((N_B, N_A, N_NBH), jnp.int32),
        ],
    )(x, nbh, W_in2f)


@functools.lru_cache(maxsize=1)
def _build_sc_gather():
    @functools.partial(
        pl.kernel,
        out_type=jax.ShapeDtypeStruct((E_TOTAL, 2 * N_FILTERS), jnp.float32),
        mesh=plsc.VectorSubcoreMesh(core_axis_name="c", subcore_axis_name="s"),
        scratch_types=[
            pltpu.VMEM((CHE,), jnp.int32),
            pltpu.VMEM((CHE, 2 * N_FILTERS), jnp.float32),
            pltpu.SemaphoreType.DMA,
        ],
    )
    def _sc_body(gidx_hbm, y_hbm, out_hbm, idx_v, rows_v, sem):
        c = lax.axis_index("c")
        s = lax.axis_index("s")
        base = (c * SC_NS + s) * E_PER_W

        def step(i, carry):
            off = base + i * CHE
            pltpu.sync_copy(gidx_hbm.at[pl.ds(off, CHE)], idx_v)
            pltpu.async_copy(y_hbm.at[idx_v], rows_v, sem).wait()
            pltpu.sync_copy(rows_v, out_hbm.at[pl.ds(off, CHE)])
            return carry

        lax.fori_loop(0, STEPS, step, 0)

    return _sc_body


def _sc_gather(gidx, y_flat):
    return _build_sc_gather()(gidx, y_flat)


def _c_body(g_ref, rij_ref, rik_ref, rjk_ref, msk_ref,
            wfd_ref, bfd_ref, wft_ref, bft_ref, wout_ref, bout_ref, out_ref):
    rij = rij_ref[0]
    rik = rik_ref[0]
    rjk = rjk_ref[0]
    msk = msk_ref[0]
    cos = (rij * rij + rik * rik - rjk * rjk) / (2.0 * rij * rik + 1e-8)
    cos = jnp.clip(cos, -1.0, 1.0)
    radial = rij * rik
    tp = 1.0 + cos
    tm = 1.0 - cos
    tp4 = (tp * tp) * (tp * tp)
    tm4 = (tm * tm) * (tm * tm)
    ang = (
        _C1 * tp,
        _C2 * tp4 * jnp.sqrt(tp),
        _C3 * tp4 * tp4,
        _C1 * tm,
        _C2 * tm4 * jnp.sqrt(tm),
        _C3 * tm4 * tm4,
    )
    wijk = bft_ref[0][None, None, :]
    for k in range(2 * N_ZETA):
        wijk = wijk + (ang[k] * radial)[..., None] * wft_ref[k][None, None, :]
    wd = rij[..., None] * wfd_ref[0][None, None, :] + bfd_ref[0][None, None, :]
    filt = jnp.concatenate([wd, wijk], axis=-1) * msk[..., None]
    s = jnp.sum(filt * g_ref[0], axis=1)
    out_ref[0] = (
        jnp.dot(s, wout_ref[...], preferred_element_type=jnp.float32)
        + bout_ref[0][None, :]
    )


def _stage_c(g, r_ij, r_ik, r_jk, msk, W_fd, b_fd, W_ft, b_ft, W_out, b_out):
    grid = (N_B, N_A // A_BLK_C)
    r_spec = pl.BlockSpec((1, A_BLK_C, N_NBH), lambda b, i: (b, i, 0))
    w_spec = lambda shape: pl.BlockSpec(shape, lambda b, i: tuple(0 for _ in shape))
    return pl.pallas_call(
        _c_body,
        grid=grid,
        in_specs=[
            pl.BlockSpec((1, A_BLK_C, N_NBH, 2 * N_FILTERS), lambda b, i: (b, i, 0, 0)),
            r_spec, r_spec, r_spec, r_spec,
            w_spec((1, N_FILTERS)),
            w_spec((1, N_FILTERS)),
            w_spec((2 * N_ZETA, N_FILTERS)),
            w_spec((1, N_FILTERS)),
            w_spec((2 * N_FILTERS, N_OUT)),
            w_spec((1, N_OUT)),
        ],
        out_specs=pl.BlockSpec((1, A_BLK_C, N_OUT), lambda b, i: (b, i, 0)),
        out_shape=jax.ShapeDtypeStruct((N_B, N_A, N_OUT), jnp.float32),
    )(g, r_ij, r_ik, r_jk, msk, W_fd, b_fd, W_ft, b_ft, W_out, b_out)


def kernel(x, r_ij, r_ik, r_jk, neighbors_j, triple_masks,
           W_in2f, W_fd, b_fd, W_ft, b_ft, W_out, b_out):
    nbh = neighbors_j.astype(jnp.int32)
    y, gidx = _stage_a(x, nbh, W_in2f)
    g = _sc_gather(gidx.reshape(E_TOTAL), y.reshape(N_B * N_A, 2 * N_FILTERS))
    g = g.reshape(N_B, N_A, N_NBH, 2 * N_FILTERS)
    return _stage_c(
        g, r_ij, r_ik, r_jk, triple_masks,
        W_fd.reshape(1, N_FILTERS), b_fd.reshape(1, N_FILTERS),
        W_ft, b_ft.reshape(1, N_FILTERS),
        W_out, b_out.reshape(1, N_OUT),
    )


# speedup vs baseline: 3583.6160x; 1.0676x over previous
import functools

import jax
import jax.numpy as jnp
from jax import lax
from jax.experimental import pallas as pl
from jax.experimental.pallas import tpu as pltpu
from jax.experimental.pallas import tpu_sc as plsc

N_B, N_A, N_NBH = 2, 10000, 16
N_IN, N_FILTERS, N_OUT = 128, 64, 128
N_ZETA = 3
_C1, _C2, _C3 = 1.0, 2.0 ** (-3.5), 2.0 ** (-7.0)

A_BLK_A = 1000
A_BLK_C = 400

SC_NC, SC_NS = 2, 16
E_TOTAL = N_B * N_A * N_NBH
E_PER_W = E_TOTAL // (SC_NC * SC_NS)
CHE = 400
STEPS = E_PER_W // CHE


def _a_body(x_ref, nbh_ref, w_ref, y_ref, gidx_ref):
    b = pl.program_id(0)
    y_ref[0] = jnp.dot(x_ref[0], w_ref[...], preferred_element_type=jnp.float32)
    gidx_ref[0] = nbh_ref[0] + b * N_A


def _stage_a(x, nbh, W_in2f):
    grid = (N_B, N_A // A_BLK_A)
    return pl.pallas_call(
        _a_body,
        grid=grid,
        in_specs=[
            pl.BlockSpec((1, A_BLK_A, N_IN), lambda b, i: (b, i, 0)),
            pl.BlockSpec((1, A_BLK_A, N_NBH), lambda b, i: (b, i, 0)),
            pl.BlockSpec((N_IN, 2 * N_FILTERS), lambda b, i: (0, 0)),
        ],
        out_specs=[
            pl.BlockSpec((1, A_BLK_A, 2 * N_FILTERS), lambda b, i: (b, i, 0)),
            pl.BlockSpec((1, A_BLK_A, N_NBH), lambda b, i: (b, i, 0)),
        ],
        out_shape=[
            jax.ShapeDtypeStruct((N_B, N_A, 2 * N_FILTERS), jnp.float32),
            jax.ShapeDtypeStruct((N_B, N_A, N_NBH), jnp.int32),
        ],
    )(x, nbh, W_in2f)


@functools.lru_cache(maxsize=1)
def _build_sc_gather():
    @functools.partial(
        pl.kernel,
        out_type=jax.ShapeDtypeStruct((E_TOTAL, 2 * N_FILTERS), jnp.float32),
        mesh=plsc.VectorSubcoreMesh(core_axis_name="c", subcore_axis_name="s"),
        scratch_types=[
            pltpu.VMEM((CHE,), jnp.int32),
            pltpu.VMEM((CHE,), jnp.int32),
            pltpu.VMEM((CHE, 2 * N_FILTERS), jnp.float32),
            pltpu.VMEM((CHE, 2 * N_FILTERS), jnp.float32),
            pltpu.SemaphoreType.DMA,
            pltpu.SemaphoreType.DMA,
        ],
    )
    def _sc_body(gidx_hbm, y_hbm, out_hbm, idx0, idx1, rows0, rows1, sem0, sem1):
        c = lax.axis_index("c")
        s = lax.axis_index("s")
        base = (c * SC_NS + s) * E_PER_W
        idx = (idx0, idx1)
        rows = (rows0, rows1)
        sem = (sem0, sem1)

        def start(i):
            b = i % 2
            off = base + i * CHE
            pltpu.sync_copy(gidx_hbm.at[pl.ds(off, CHE)], idx[b])
            return pltpu.async_copy(y_hbm.at[idx[b]], rows[b], sem[b])

        handles = [start(0)]
        for i in range(STEPS):
            if i + 1 < STEPS:
                handles.append(start(i + 1))
            handles[i].wait()
            off = base + i * CHE
            pltpu.sync_copy(rows[i % 2], out_hbm.at[pl.ds(off, CHE)])

    return _sc_body


def _sc_gather(gidx, y_flat):
    return _build_sc_gather()(gidx, y_flat)


def _c_body(g_ref, rij_ref, rik_ref, rjk_ref, msk_ref,
            wrows_ref, wout_ref, bout_ref, out_ref):
    rij = rij_ref[0]
    rik = rik_ref[0]
    rjk = rjk_ref[0]
    msk = msk_ref[0]
    cos = (rij * rij + rik * rik - rjk * rjk) / (2.0 * rij * rik + 1e-8)
    cos = jnp.clip(cos, -1.0, 1.0)
    radial = msk * rij * rik
    tp = 1.0 + cos
    tm = 1.0 - cos
    tp4 = (tp * tp) * (tp * tp)
    tm4 = (tm * tm) * (tm * tm)
    coeffs = (
        msk * rij,
        msk,
        _C1 * radial * tp,
        _C2 * radial * tp4 * jnp.sqrt(tp),
        _C3 * radial * tp4 * tp4,
        _C1 * radial * tm,
        _C2 * radial * tm4 * jnp.sqrt(tm),
        _C3 * radial * tm4 * tm4,
    )
    filt = coeffs[0][..., None] * wrows_ref[0][None, None, :]
    for j in range(1, 8):
        filt = filt + coeffs[j][..., None] * wrows_ref[j][None, None, :]
    s = jnp.sum(filt * g_ref[0], axis=1)
    out_ref[0] = (
        jnp.dot(s, wout_ref[...], preferred_element_type=jnp.float32)
        + bout_ref[0][None, :]
    )


def _stage_c(g, r_ij, r_ik, r_jk, msk, wrows, W_out, b_out):
    grid = (N_B, N_A // A_BLK_C)
    r_spec = pl.BlockSpec((1, A_BLK_C, N_NBH), lambda b, i: (b, i, 0))
    w_spec = lambda shape: pl.BlockSpec(shape, lambda b, i: tuple(0 for _ in shape))
    return pl.pallas_call(
        _c_body,
        grid=grid,
        in_specs=[
            pl.BlockSpec((1, A_BLK_C, N_NBH, 2 * N_FILTERS), lambda b, i: (b, i, 0, 0)),
            r_spec, r_spec, r_spec, r_spec,
            w_spec((8, 2 * N_FILTERS)),
            w_spec((2 * N_FILTERS, N_OUT)),
            w_spec((1, N_OUT)),
        ],
        out_specs=pl.BlockSpec((1, A_BLK_C, N_OUT), lambda b, i: (b, i, 0)),
        out_shape=jax.ShapeDtypeStruct((N_B, N_A, N_OUT), jnp.float32),
    )(g, r_ij, r_ik, r_jk, msk, wrows, W_out, b_out)


def kernel(x, r_ij, r_ik, r_jk, neighbors_j, triple_masks,
           W_in2f, W_fd, b_fd, W_ft, b_ft, W_out, b_out):
    nbh = neighbors_j.astype(jnp.int32)
    y, gidx = _stage_a(x, nbh, W_in2f)
    g = _sc_gather(gidx.reshape(E_TOTAL), y.reshape(N_B * N_A, 2 * N_FILTERS))
    g = g.reshape(N_B, N_A, N_NBH, 2 * N_FILTERS)
    zeros = jnp.zeros((N_FILTERS,), jnp.float32)
    wrows = jnp.stack([
        jnp.concatenate([W_fd[0], zeros]),
        jnp.concatenate([b_fd, b_ft]),
        jnp.concatenate([zeros, W_ft[0]]),
        jnp.concatenate([zeros, W_ft[1]]),
        jnp.concatenate([zeros, W_ft[2]]),
        jnp.concatenate([zeros, W_ft[3]]),
        jnp.concatenate([zeros, W_ft[4]]),
        jnp.concatenate([zeros, W_ft[5]]),
    ])
    return _stage_c(
        g, r_ij, r_ik, r_jk, triple_masks,
        wrows, W_out, b_out.reshape(1, N_OUT),
    )
